# Initial kernel scaffold; baseline (speedup 1.0000x reference)
import jax
import jax.numpy as jnp
from jax.experimental import pallas as pl


def kernel(x, edge_index, W1l, b1l, W1r, g1, be1, W2l, b2l, W2r, g2, be2, Wc, bc):
    raise NotImplementedError("write your pallas kernel here")


# speedup vs baseline: 5.1804x; 5.1804x over previous
import functools

import jax
import jax.numpy as jnp
from jax import lax
from jax.experimental import pallas as pl
from jax.experimental.pallas import tpu as pltpu
from jax.experimental.pallas import tpu_sc as plsc

_N = 10000
_E = 320000
_D = 128
_AUG = 144

_NC = 2
_NS = 16
_NW = _NC * _NS
_EPW = _E // _NW
_CHUNK = 80
_STRIPE = 624
_TAIL = _N - _NS * _STRIPE


@functools.lru_cache(maxsize=None)
def _make_sc_agg(width):
    mesh = plsc.VectorSubcoreMesh(core_axis_name="c", subcore_axis_name="s")

    @functools.partial(
        pl.kernel,
        out_type=jax.ShapeDtypeStruct((_NC, _N, width), jnp.float32),
        mesh=mesh,
        scratch_types=[
            pltpu.VMEM((_CHUNK,), jnp.int32),
            pltpu.VMEM((_CHUNK,), jnp.int32),
            pltpu.VMEM((_CHUNK, width), jnp.float32),
            pltpu.VMEM_SHARED((_N, width), jnp.float32),
            pltpu.SemaphoreType.DMA,
        ],
        compiler_params=pltpu.CompilerParams(use_tc_tiling_on_sc=False),
    )
    def sc_agg(h_hbm, src_hbm, dst_hbm, zero_hbm, out_hbm,
               src_v, dst_v, rows_v, acc_sh, sem):
        cid = lax.axis_index("c")
        sid = lax.axis_index("s")
        wid = cid * _NS + sid

        pltpu.sync_copy(zero_hbm.at[pl.ds(sid * _STRIPE, _STRIPE)],
                        acc_sh.at[pl.ds(sid * _STRIPE, _STRIPE)])

        @pl.when(sid == 0)
        def _():
            pltpu.sync_copy(zero_hbm.at[pl.ds(_NS * _STRIPE, _TAIL)],
                            acc_sh.at[pl.ds(_NS * _STRIPE, _TAIL)])

        plsc.subcore_barrier()

        base = wid * _EPW

        def body(i, carry):
            off = base + i * _CHUNK
            pltpu.sync_copy(src_hbm.at[pl.ds(off, _CHUNK)], src_v)
            pltpu.sync_copy(dst_hbm.at[pl.ds(off, _CHUNK)], dst_v)
            pltpu.async_copy(h_hbm.at[src_v], rows_v, sem).wait()
            pltpu.sync_copy(rows_v, acc_sh.at[dst_v], add=True)
            return carry

        lax.fori_loop(0, _EPW // _CHUNK, body, 0)

        plsc.subcore_barrier()

        pltpu.sync_copy(acc_sh.at[pl.ds(sid * _STRIPE, _STRIPE)],
                        out_hbm.at[cid, pl.ds(sid * _STRIPE, _STRIPE)])

        @pl.when(sid == 0)
        def _():
            pltpu.sync_copy(acc_sh.at[pl.ds(_NS * _STRIPE, _TAIL)],
                            out_hbm.at[cid, pl.ds(_NS * _STRIPE, _TAIL)])

    return sc_agg


def _dense1_body(acc_ref, x_ref, wl_ref, bl_ref, wr_ref, g_ref, be_ref,
                 h_ref, cnt_ref):
    s = acc_ref[0, :, :_D] + acc_ref[1, :, :_D]
    cnt = acc_ref[0, :, _D:_D + 1] + acc_ref[1, :, _D:_D + 1]
    mean = s / jnp.maximum(cnt, 1.0)
    z = (lax.dot_general(mean, wl_ref[...], (((1,), (1,)), ((), ())),
                         preferred_element_type=jnp.float32)
         + lax.dot_general(x_ref[...], wr_ref[...], (((1,), (1,)), ((), ())),
                           preferred_element_type=jnp.float32)
         + bl_ref[...])
    mu = jnp.mean(z, axis=0)
    var = jnp.mean((z - mu) ** 2, axis=0)
    zn = (z - mu) / jnp.sqrt(var + 1e-5) * g_ref[...] + be_ref[...]
    h_ref[...] = jnp.maximum(zn, 0.0)
    cnt_ref[...] = cnt


def _dense2_body(acc_ref, cnt_ref, h_ref, wl_ref, bl_ref, wr_ref, g_ref,
                 be_ref, wc_ref, bc_ref, out_ref):
    s = acc_ref[0] + acc_ref[1]
    mean = s / jnp.maximum(cnt_ref[...], 1.0)
    z = (lax.dot_general(mean, wl_ref[...], (((1,), (1,)), ((), ())),
                         preferred_element_type=jnp.float32)
         + lax.dot_general(h_ref[...], wr_ref[...], (((1,), (1,)), ((), ())),
                           preferred_element_type=jnp.float32)
         + bl_ref[...])
    mu = jnp.mean(z, axis=0)
    var = jnp.mean((z - mu) ** 2, axis=0)
    zn = (z - mu) / jnp.sqrt(var + 1e-5) * g_ref[...] + be_ref[...]
    h2 = jnp.maximum(zn, 0.0)
    out_ref[...] = (lax.dot_general(h2, wc_ref[...], (((1,), (1,)), ((), ())),
                                    preferred_element_type=jnp.float32)
                    + bc_ref[...])


def kernel(x, edge_index, W1l, b1l, W1r, g1, be1, W2l, b2l, W2r, g2, be2,
           Wc, bc):
    src = edge_index[0]
    dst = edge_index[1]

    ones_col = jnp.ones((_N, 1), jnp.float32)
    pad = jnp.zeros((_N, _AUG - _D - 1), jnp.float32)
    x_aug = jnp.concatenate([x, ones_col, pad], axis=1)
    zero_aug = jnp.zeros((_N, _AUG), jnp.float32)
    zero_d = jnp.zeros((_N, _D), jnp.float32)

    acc1 = _make_sc_agg(_AUG)(x_aug, src, dst, zero_aug)

    h1, cnt = pl.pallas_call(
        _dense1_body,
        out_shape=(jax.ShapeDtypeStruct((_N, _D), jnp.float32),
                   jax.ShapeDtypeStruct((_N, 1), jnp.float32)),
    )(acc1, x, W1l, b1l, W1r, g1, be1)

    acc2 = _make_sc_agg(_D)(h1, src, dst, zero_d)

    n_cls = Wc.shape[0]
    Wc_pad = jnp.zeros((_D, _D), jnp.float32).at[:n_cls].set(Wc)
    bc_pad = jnp.zeros((_D,), jnp.float32).at[:n_cls].set(bc)
    logits_pad = pl.pallas_call(
        _dense2_body,
        out_shape=jax.ShapeDtypeStruct((_N, _D), jnp.float32),
    )(acc2, cnt, h1, W2l, b2l, W2r, g2, be2, Wc_pad, bc_pad)

    return logits_pad[:, :n_cls]


# speedup vs baseline: 10.2823x; 1.9848x over previous
import functools

import jax
import jax.numpy as jnp
from jax import lax
from jax.experimental import pallas as pl
from jax.experimental.pallas import tpu as pltpu
from jax.experimental.pallas import tpu_sc as plsc

_N = 10000
_E = 320000
_D = 128
_AUG = 144

_NC = 2
_NS = 16
_NW = _NC * _NS
_EPW = _E // _NW
_CHUNK = 80
_NCH = _EPW // _CHUNK
_SB = 25
_NB = 3
_STRIPE = 624
_TAIL = _N - _NS * _STRIPE


@functools.lru_cache(maxsize=None)
def _make_sc_agg(width):
    mesh = plsc.VectorSubcoreMesh(core_axis_name="c", subcore_axis_name="s")

    @functools.partial(
        pl.kernel,
        out_type=jax.ShapeDtypeStruct((_NC, _N, width), jnp.float32),
        mesh=mesh,
        scratch_types=[
            pltpu.VMEM((_SB, _CHUNK), jnp.int32),
            pltpu.VMEM((_SB, _CHUNK), jnp.int32),
            pltpu.VMEM((_NB, _CHUNK, width), jnp.float32),
            pltpu.VMEM_SHARED((_N, width), jnp.float32),
            pltpu.SemaphoreType.DMA,
        ],
        compiler_params=pltpu.CompilerParams(use_tc_tiling_on_sc=False),
    )
    def sc_agg(h_hbm, src_hbm, dst_hbm, zero_hbm, out_hbm,
               src_v, dst_v, rows_v, acc_sh, sem):
        cid = lax.axis_index("c")
        sid = lax.axis_index("s")
        wid = cid * _NS + sid

        pltpu.sync_copy(zero_hbm.at[pl.ds(sid * _STRIPE, _STRIPE)],
                        acc_sh.at[pl.ds(sid * _STRIPE, _STRIPE)])

        @pl.when(sid == 0)
        def _():
            pltpu.sync_copy(zero_hbm.at[pl.ds(_NS * _STRIPE, _TAIL)],
                            acc_sh.at[pl.ds(_NS * _STRIPE, _TAIL)])

        plsc.subcore_barrier()

        def superblock(sb, carry):
            pltpu.sync_copy(src_hbm.at[wid, pl.ds(sb * _SB, _SB)], src_v)
            pltpu.sync_copy(dst_hbm.at[wid, pl.ds(sb * _SB, _SB)], dst_v)

            for b in range(_NB - 1):
                pltpu.async_copy(h_hbm.at[src_v.at[b]], rows_v.at[b], sem)

            def body(i, c):
                buf = lax.rem(i, _NB)
                pltpu.make_async_copy(h_hbm.at[src_v.at[i]], rows_v.at[buf],
                                      sem).wait()
                pltpu.sync_copy(rows_v.at[buf], acc_sh.at[dst_v.at[i]],
                                add=True)

                @pl.when(i + _NB - 1 < _SB)
                def _():
                    nxt = lax.rem(i + _NB - 1, _NB)
                    pltpu.async_copy(h_hbm.at[src_v.at[i + _NB - 1]],
                                     rows_v.at[nxt], sem)

                return c

            lax.fori_loop(0, _SB, body, 0)
            return carry

        lax.fori_loop(0, _NCH // _SB, superblock, 0)

        plsc.subcore_barrier()

        pltpu.sync_copy(acc_sh.at[pl.ds(sid * _STRIPE, _STRIPE)],
                        out_hbm.at[cid, pl.ds(sid * _STRIPE, _STRIPE)])

        @pl.when(sid == 0)
        def _():
            pltpu.sync_copy(acc_sh.at[pl.ds(_NS * _STRIPE, _TAIL)],
                            out_hbm.at[cid, pl.ds(_NS * _STRIPE, _TAIL)])

    return sc_agg


def _dense1_body(acc_ref, x_ref, wl_ref, bl_ref, wr_ref, g_ref, be_ref,
                 h_ref, cnt_ref):
    s = acc_ref[0, :, :_D] + acc_ref[1, :, :_D]
    cnt = acc_ref[0, :, _D:_D + 1] + acc_ref[1, :, _D:_D + 1]
    mean = s / jnp.maximum(cnt, 1.0)
    z = (lax.dot_general(mean, wl_ref[...], (((1,), (1,)), ((), ())),
                         preferred_element_type=jnp.float32)
         + lax.dot_general(x_ref[...], wr_ref[...], (((1,), (1,)), ((), ())),
                           preferred_element_type=jnp.float32)
         + bl_ref[...])
    mu = jnp.mean(z, axis=0)
    var = jnp.mean((z - mu) ** 2, axis=0)
    zn = (z - mu) / jnp.sqrt(var + 1e-5) * g_ref[...] + be_ref[...]
    h_ref[...] = jnp.maximum(zn, 0.0)
    cnt_ref[...] = cnt


def _dense2_body(acc_ref, cnt_ref, h_ref, wl_ref, bl_ref, wr_ref, g_ref,
                 be_ref, wc_ref, bc_ref, out_ref):
    s = acc_ref[0] + acc_ref[1]
    mean = s / jnp.maximum(cnt_ref[...], 1.0)
    z = (lax.dot_general(mean, wl_ref[...], (((1,), (1,)), ((), ())),
                         preferred_element_type=jnp.float32)
         + lax.dot_general(h_ref[...], wr_ref[...], (((1,), (1,)), ((), ())),
                           preferred_element_type=jnp.float32)
         + bl_ref[...])
    mu = jnp.mean(z, axis=0)
    var = jnp.mean((z - mu) ** 2, axis=0)
    zn = (z - mu) / jnp.sqrt(var + 1e-5) * g_ref[...] + be_ref[...]
    h2 = jnp.maximum(zn, 0.0)
    out_ref[...] = (lax.dot_general(h2, wc_ref[...], (((1,), (1,)), ((), ())),
                                    preferred_element_type=jnp.float32)
                    + bc_ref[...])


def kernel(x, edge_index, W1l, b1l, W1r, g1, be1, W2l, b2l, W2r, g2, be2,
           Wc, bc):
    src = edge_index[0].reshape(_NW, _NCH, _CHUNK)
    dst = edge_index[1].reshape(_NW, _NCH, _CHUNK)

    ones_col = jnp.ones((_N, 1), jnp.float32)
    pad = jnp.zeros((_N, _AUG - _D - 1), jnp.float32)
    x_aug = jnp.concatenate([x, ones_col, pad], axis=1)
    zero_aug = jnp.zeros((_N, _AUG), jnp.float32)
    zero_d = jnp.zeros((_N, _D), jnp.float32)

    acc1 = _make_sc_agg(_AUG)(x_aug, src, dst, zero_aug)

    h1, cnt = pl.pallas_call(
        _dense1_body,
        out_shape=(jax.ShapeDtypeStruct((_N, _D), jnp.float32),
                   jax.ShapeDtypeStruct((_N, 1), jnp.float32)),
    )(acc1, x, W1l, b1l, W1r, g1, be1)

    acc2 = _make_sc_agg(_D)(h1, src, dst, zero_d)

    n_cls = Wc.shape[0]
    Wc_pad = jnp.zeros((_D, _D), jnp.float32).at[:n_cls].set(Wc)
    bc_pad = jnp.zeros((_D,), jnp.float32).at[:n_cls].set(bc)
    logits_pad = pl.pallas_call(
        _dense2_body,
        out_shape=jax.ShapeDtypeStruct((_N, _D), jnp.float32),
    )(acc2, cnt, h1, W2l, b2l, W2r, g2, be2, Wc_pad, bc_pad)

    return logits_pad[:, :n_cls]


# speedup vs baseline: 11.4562x; 1.1142x over previous
import functools

import jax
import jax.numpy as jnp
from jax import lax
from jax.experimental import pallas as pl
from jax.experimental.pallas import tpu as pltpu
from jax.experimental.pallas import tpu_sc as plsc

_N = 10000
_E = 320000
_D = 128
_AUG = 144

_NC = 2
_NS = 16
_NW = _NC * _NS
_EPW = _E // _NW
_CHUNK = 80
_NCH = _EPW // _CHUNK
_SB = 25
_NB = 3
_STRIPE = 624
_TAIL = _N - _NS * _STRIPE


@functools.lru_cache(maxsize=None)
def _make_sc_agg(width):
    mesh = plsc.VectorSubcoreMesh(core_axis_name="c", subcore_axis_name="s")

    @functools.partial(
        pl.kernel,
        out_type=jax.ShapeDtypeStruct((_NC, _N, width), jnp.float32),
        mesh=mesh,
        scratch_types=[
            pltpu.VMEM((_SB, _CHUNK), jnp.int32),
            pltpu.VMEM((_SB, _CHUNK), jnp.int32),
            pltpu.VMEM((_NB, _CHUNK, width), jnp.float32),
            pltpu.VMEM_SHARED((_N, width), jnp.float32),
            pltpu.SemaphoreType.DMA,
            pltpu.SemaphoreType.DMA,
        ],
        compiler_params=pltpu.CompilerParams(use_tc_tiling_on_sc=False),
    )
    def sc_agg(h_hbm, src_hbm, dst_hbm, zero_hbm, out_hbm,
               src_v, dst_v, rows_v, acc_sh, sem, sem_s):
        cid = lax.axis_index("c")
        sid = lax.axis_index("s")
        wid = cid * _NS + sid

        pltpu.sync_copy(zero_hbm.at[pl.ds(sid * _STRIPE, _STRIPE)],
                        acc_sh.at[pl.ds(sid * _STRIPE, _STRIPE)])

        @pl.when(sid == 0)
        def _():
            pltpu.sync_copy(zero_hbm.at[pl.ds(_NS * _STRIPE, _TAIL)],
                            acc_sh.at[pl.ds(_NS * _STRIPE, _TAIL)])

        plsc.subcore_barrier()

        def superblock(sb, carry):
            pltpu.sync_copy(src_hbm.at[wid, pl.ds(sb * _SB, _SB)], src_v)
            pltpu.sync_copy(dst_hbm.at[wid, pl.ds(sb * _SB, _SB)], dst_v)

            for b in range(_NB - 1):
                pltpu.async_copy(h_hbm.at[src_v.at[b]], rows_v.at[b], sem)

            def body(i, c):
                buf = lax.rem(i, _NB)
                pltpu.make_async_copy(h_hbm.at[src_v.at[i]], rows_v.at[buf],
                                      sem).wait()
                pltpu.async_copy(rows_v.at[buf], acc_sh.at[dst_v.at[i]],
                                 sem_s, add=True)

                @pl.when(i > 0)
                def _():
                    pltpu.make_async_copy(rows_v.at[buf],
                                          acc_sh.at[dst_v.at[i]],
                                          sem_s).wait()

                @pl.when(i + _NB - 1 < _SB)
                def _():
                    nxt = lax.rem(i + _NB - 1, _NB)
                    pltpu.async_copy(h_hbm.at[src_v.at[i + _NB - 1]],
                                     rows_v.at[nxt], sem)

                return c

            lax.fori_loop(0, _SB, body, 0)
            pltpu.make_async_copy(rows_v.at[0], acc_sh.at[dst_v.at[0]],
                                  sem_s).wait()
            return carry

        lax.fori_loop(0, _NCH // _SB, superblock, 0)

        plsc.subcore_barrier()

        pltpu.sync_copy(acc_sh.at[pl.ds(sid * _STRIPE, _STRIPE)],
                        out_hbm.at[cid, pl.ds(sid * _STRIPE, _STRIPE)])

        @pl.when(sid == 0)
        def _():
            pltpu.sync_copy(acc_sh.at[pl.ds(_NS * _STRIPE, _TAIL)],
                            out_hbm.at[cid, pl.ds(_NS * _STRIPE, _TAIL)])

    return sc_agg


def _dense1_body(acc_ref, x_ref, wl_ref, bl_ref, wr_ref, g_ref, be_ref,
                 h_ref, cnt_ref):
    s = acc_ref[0, :, :_D] + acc_ref[1, :, :_D]
    cnt = acc_ref[0, :, _D:_D + 1] + acc_ref[1, :, _D:_D + 1]
    mean = s / jnp.maximum(cnt, 1.0)
    z = (lax.dot_general(mean, wl_ref[...], (((1,), (1,)), ((), ())),
                         preferred_element_type=jnp.float32)
         + lax.dot_general(x_ref[...], wr_ref[...], (((1,), (1,)), ((), ())),
                           preferred_element_type=jnp.float32)
         + bl_ref[...])
    mu = jnp.mean(z, axis=0)
    var = jnp.mean((z - mu) ** 2, axis=0)
    zn = (z - mu) / jnp.sqrt(var + 1e-5) * g_ref[...] + be_ref[...]
    h_ref[...] = jnp.maximum(zn, 0.0)
    cnt_ref[...] = cnt


def _dense2_body(acc_ref, cnt_ref, h_ref, wl_ref, bl_ref, wr_ref, g_ref,
                 be_ref, wc_ref, bc_ref, out_ref):
    s = acc_ref[0] + acc_ref[1]
    mean = s / jnp.maximum(cnt_ref[...], 1.0)
    z = (lax.dot_general(mean, wl_ref[...], (((1,), (1,)), ((), ())),
                         preferred_element_type=jnp.float32)
         + lax.dot_general(h_ref[...], wr_ref[...], (((1,), (1,)), ((), ())),
                           preferred_element_type=jnp.float32)
         + bl_ref[...])
    mu = jnp.mean(z, axis=0)
    var = jnp.mean((z - mu) ** 2, axis=0)
    zn = (z - mu) / jnp.sqrt(var + 1e-5) * g_ref[...] + be_ref[...]
    h2 = jnp.maximum(zn, 0.0)
    out_ref[...] = (lax.dot_general(h2, wc_ref[...], (((1,), (1,)), ((), ())),
                                    preferred_element_type=jnp.float32)
                    + bc_ref[...])


def kernel(x, edge_index, W1l, b1l, W1r, g1, be1, W2l, b2l, W2r, g2, be2,
           Wc, bc):
    src = edge_index[0].reshape(_NW, _NCH, _CHUNK)
    dst = edge_index[1].reshape(_NW, _NCH, _CHUNK)

    ones_col = jnp.ones((_N, 1), jnp.float32)
    pad = jnp.zeros((_N, _AUG - _D - 1), jnp.float32)
    x_aug = jnp.concatenate([x, ones_col, pad], axis=1)
    zero_aug = jnp.zeros((_N, _AUG), jnp.float32)
    zero_d = jnp.zeros((_N, _D), jnp.float32)

    acc1 = _make_sc_agg(_AUG)(x_aug, src, dst, zero_aug)

    h1, cnt = pl.pallas_call(
        _dense1_body,
        out_shape=(jax.ShapeDtypeStruct((_N, _D), jnp.float32),
                   jax.ShapeDtypeStruct((_N, 1), jnp.float32)),
    )(acc1, x, W1l, b1l, W1r, g1, be1)

    acc2 = _make_sc_agg(_D)(h1, src, dst, zero_d)

    n_cls = Wc.shape[0]
    Wc_pad = jnp.zeros((_D, _D), jnp.float32).at[:n_cls].set(Wc)
    bc_pad = jnp.zeros((_D,), jnp.float32).at[:n_cls].set(bc)
    logits_pad = pl.pallas_call(
        _dense2_body,
        out_shape=jax.ShapeDtypeStruct((_N, _D), jnp.float32),
    )(acc2, cnt, h1, W2l, b2l, W2r, g2, be2, Wc_pad, bc_pad)

    return logits_pad[:, :n_cls]


# speedup vs baseline: 12.8904x; 1.1252x over previous
import functools

import jax
import jax.numpy as jnp
from jax import lax
from jax.experimental import pallas as pl
from jax.experimental.pallas import tpu as pltpu
from jax.experimental.pallas import tpu_sc as plsc

_N = 10000
_E = 320000
_D = 128
_CW = 16

_NC = 2
_NS = 16
_NW = _NC * _NS
_EPW = _E // _NW
_CHUNK = 80
_NCH = _EPW // _CHUNK
_SB = 25
_NB = 3
_STRIPE = 624
_TAIL = _N - _NS * _STRIPE


@functools.lru_cache(maxsize=None)
def _make_sc_agg(with_cnt):
    mesh = plsc.VectorSubcoreMesh(core_axis_name="c", subcore_axis_name="s")

    out_type = [jax.ShapeDtypeStruct((_NC, _N, _D), jnp.float32)]
    scratch = [
        pltpu.VMEM((_SB, _CHUNK), jnp.int32),
        pltpu.VMEM((_SB, _CHUNK), jnp.int32),
        pltpu.VMEM((_NB, _CHUNK, _D), jnp.float32),
        pltpu.VMEM_SHARED((_N, _D), jnp.float32),
        pltpu.SemaphoreType.DMA,
        pltpu.SemaphoreType.DMA,
    ]
    if with_cnt:
        out_type.append(jax.ShapeDtypeStruct((_NC, _N, _CW), jnp.float32))
        scratch += [
            pltpu.VMEM((_CHUNK, _CW), jnp.float32),
            pltpu.VMEM_SHARED((_N, _CW), jnp.float32),
            pltpu.SemaphoreType.DMA,
        ]

    @functools.partial(
        pl.kernel,
        out_type=tuple(out_type) if with_cnt else out_type[0],
        mesh=mesh,
        scratch_types=scratch,
        compiler_params=pltpu.CompilerParams(use_tc_tiling_on_sc=False),
    )
    def sc_agg(h_hbm, src_hbm, dst_hbm, zero_hbm, *rest):
        if with_cnt:
            (onehot_hbm, zero16_hbm, out_hbm, cnt_hbm,
             src_v, dst_v, rows_v, acc_sh, sem, sem_s,
             one_v, cacc_sh, sem_c) = rest
        else:
            (out_hbm, src_v, dst_v, rows_v, acc_sh, sem, sem_s) = rest

        cid = lax.axis_index("c")
        sid = lax.axis_index("s")
        wid = cid * _NS + sid

        pltpu.sync_copy(zero_hbm.at[pl.ds(sid * _STRIPE, _STRIPE)],
                        acc_sh.at[pl.ds(sid * _STRIPE, _STRIPE)])
        if with_cnt:
            pltpu.sync_copy(zero16_hbm.at[pl.ds(sid * _STRIPE, _STRIPE)],
                            cacc_sh.at[pl.ds(sid * _STRIPE, _STRIPE)])
            pltpu.sync_copy(onehot_hbm, one_v)

        @pl.when(sid == 0)
        def _():
            pltpu.sync_copy(zero_hbm.at[pl.ds(_NS * _STRIPE, _TAIL)],
                            acc_sh.at[pl.ds(_NS * _STRIPE, _TAIL)])
            if with_cnt:
                pltpu.sync_copy(zero16_hbm.at[pl.ds(_NS * _STRIPE, _TAIL)],
                                cacc_sh.at[pl.ds(_NS * _STRIPE, _TAIL)])

        plsc.subcore_barrier()

        def superblock(sb, carry):
            pltpu.sync_copy(src_hbm.at[wid, pl.ds(sb * _SB, _SB)], src_v)
            pltpu.sync_copy(dst_hbm.at[wid, pl.ds(sb * _SB, _SB)], dst_v)

            for b in range(_NB - 1):
                pltpu.async_copy(h_hbm.at[src_v.at[b]], rows_v.at[b], sem)

            def body(i, c):
                buf = lax.rem(i, _NB)
                pltpu.make_async_copy(h_hbm.at[src_v.at[i]], rows_v.at[buf],
                                      sem).wait()
                pltpu.async_copy(rows_v.at[buf], acc_sh.at[dst_v.at[i]],
                                 sem_s, add=True)
                if with_cnt:
                    pltpu.async_copy(one_v, cacc_sh.at[dst_v.at[i]],
                                     sem_c, add=True)

                @pl.when(i > 0)
                def _():
                    pltpu.make_async_copy(rows_v.at[buf],
                                          acc_sh.at[dst_v.at[i]],
                                          sem_s).wait()
                    if with_cnt:
                        pltpu.make_async_copy(one_v, cacc_sh.at[dst_v.at[i]],
                                              sem_c).wait()

                @pl.when(i + _NB - 1 < _SB)
                def _():
                    nxt = lax.rem(i + _NB - 1, _NB)
                    pltpu.async_copy(h_hbm.at[src_v.at[i + _NB - 1]],
                                     rows_v.at[nxt], sem)

                return c

            lax.fori_loop(0, _SB, body, 0)
            pltpu.make_async_copy(rows_v.at[0], acc_sh.at[dst_v.at[0]],
                                  sem_s).wait()
            if with_cnt:
                pltpu.make_async_copy(one_v, cacc_sh.at[dst_v.at[0]],
                                      sem_c).wait()
            return carry

        lax.fori_loop(0, _NCH // _SB, superblock, 0)

        plsc.subcore_barrier()

        pltpu.sync_copy(acc_sh.at[pl.ds(sid * _STRIPE, _STRIPE)],
                        out_hbm.at[cid, pl.ds(sid * _STRIPE, _STRIPE)])
        if with_cnt:
            pltpu.sync_copy(cacc_sh.at[pl.ds(sid * _STRIPE, _STRIPE)],
                            cnt_hbm.at[cid, pl.ds(sid * _STRIPE, _STRIPE)])

        @pl.when(sid == 0)
        def _():
            pltpu.sync_copy(acc_sh.at[pl.ds(_NS * _STRIPE, _TAIL)],
                            out_hbm.at[cid, pl.ds(_NS * _STRIPE, _TAIL)])
            if with_cnt:
                pltpu.sync_copy(cacc_sh.at[pl.ds(_NS * _STRIPE, _TAIL)],
                                cnt_hbm.at[cid, pl.ds(_NS * _STRIPE, _TAIL)])

    return sc_agg


def _dense1_body(acc_ref, cacc_ref, x_ref, wl_ref, bl_ref, wr_ref, g_ref,
                 be_ref, h_ref, cnt_ref):
    s = acc_ref[0] + acc_ref[1]
    cnt = cacc_ref[0, :, 0:1] + cacc_ref[1, :, 0:1]
    mean = s / jnp.maximum(cnt, 1.0)
    z = (lax.dot_general(mean, wl_ref[...], (((1,), (1,)), ((), ())),
                         preferred_element_type=jnp.float32)
         + lax.dot_general(x_ref[...], wr_ref[...], (((1,), (1,)), ((), ())),
                           preferred_element_type=jnp.float32)
         + bl_ref[...])
    mu = jnp.mean(z, axis=0)
    var = jnp.mean((z - mu) ** 2, axis=0)
    zn = (z - mu) / jnp.sqrt(var + 1e-5) * g_ref[...] + be_ref[...]
    h_ref[...] = jnp.maximum(zn, 0.0)
    cnt_ref[...] = cnt


def _dense2_body(acc_ref, cnt_ref, h_ref, wl_ref, bl_ref, wr_ref, g_ref,
                 be_ref, wc_ref, bc_ref, out_ref):
    s = acc_ref[0] + acc_ref[1]
    mean = s / jnp.maximum(cnt_ref[...], 1.0)
    z = (lax.dot_general(mean, wl_ref[...], (((1,), (1,)), ((), ())),
                         preferred_element_type=jnp.float32)
         + lax.dot_general(h_ref[...], wr_ref[...], (((1,), (1,)), ((), ())),
                           preferred_element_type=jnp.float32)
         + bl_ref[...])
    mu = jnp.mean(z, axis=0)
    var = jnp.mean((z - mu) ** 2, axis=0)
    zn = (z - mu) / jnp.sqrt(var + 1e-5) * g_ref[...] + be_ref[...]
    h2 = jnp.maximum(zn, 0.0)
    logits = (lax.dot_general(h2, wc_ref[...], (((1,), (1,)), ((), ())),
                              preferred_element_type=jnp.float32)
              + bc_ref[...])
    out_ref[...] = logits


def kernel(x, edge_index, W1l, b1l, W1r, g1, be1, W2l, b2l, W2r, g2, be2,
           Wc, bc):
    src = edge_index[0].reshape(_NW, _NCH, _CHUNK)
    dst = edge_index[1].reshape(_NW, _NCH, _CHUNK)

    zero_d = jnp.zeros((_N, _D), jnp.float32)
    zero16 = jnp.zeros((_N, _CW), jnp.float32)
    onehot = jnp.zeros((_CHUNK, _CW), jnp.float32).at[:, 0].set(1.0)

    acc1, cacc = _make_sc_agg(True)(x, src, dst, zero_d, onehot, zero16)

    h1, cnt = pl.pallas_call(
        _dense1_body,
        out_shape=(jax.ShapeDtypeStruct((_N, _D), jnp.float32),
                   jax.ShapeDtypeStruct((_N, 1), jnp.float32)),
    )(acc1, cacc, x, W1l, b1l, W1r, g1, be1)

    acc2 = _make_sc_agg(False)(h1, src, dst, zero_d)

    n_cls = Wc.shape[0]
    Wc_pad = jnp.zeros((_D, _D), jnp.float32).at[:n_cls].set(Wc)
    bc_pad = jnp.zeros((_D,), jnp.float32).at[:n_cls].set(bc)
    logits_pad = pl.pallas_call(
        _dense2_body,
        out_shape=jax.ShapeDtypeStruct((_N, _D), jnp.float32),
    )(acc2, cnt, h1, W2l, b2l, W2r, g2, be2, Wc_pad, bc_pad)
    return logits_pad[:, :n_cls]


# speedup vs baseline: 12.8925x; 1.0002x over previous
import functools

import jax
import jax.numpy as jnp
from jax import lax
from jax.experimental import pallas as pl
from jax.experimental.pallas import tpu as pltpu
from jax.experimental.pallas import tpu_sc as plsc

_N = 10000
_E = 320000
_D = 128
_CW = 16

_NC = 2
_NS = 16
_NW = _NC * _NS
_EPW = _E // _NW
_CHUNK = 80
_NCH = _EPW // _CHUNK
_SB = 25
_NB = 3
_STRIPE = 624
_TAIL = _N - _NS * _STRIPE


@functools.lru_cache(maxsize=None)
def _make_sc_agg(with_cnt):
    mesh = plsc.VectorSubcoreMesh(core_axis_name="c", subcore_axis_name="s")

    out_type = [jax.ShapeDtypeStruct((_NC, _N, _D), jnp.float32)]
    scratch = [
        pltpu.VMEM((_SB, _CHUNK), jnp.int32),
        pltpu.VMEM((_SB, _CHUNK), jnp.int32),
        pltpu.VMEM((_NB, _CHUNK, _D), jnp.float32),
        pltpu.VMEM_SHARED((_N, _D), jnp.float32),
        pltpu.SemaphoreType.DMA,
        pltpu.SemaphoreType.DMA,
    ]
    if with_cnt:
        out_type.append(jax.ShapeDtypeStruct((_NC, _N, _CW), jnp.float32))
        scratch += [
            pltpu.VMEM((_CHUNK, _CW), jnp.float32),
            pltpu.VMEM_SHARED((_N, _CW), jnp.float32),
            pltpu.SemaphoreType.DMA,
        ]

    @functools.partial(
        pl.kernel,
        out_type=tuple(out_type) if with_cnt else out_type[0],
        mesh=mesh,
        scratch_types=scratch,
        compiler_params=pltpu.CompilerParams(use_tc_tiling_on_sc=False),
    )
    def sc_agg(h_hbm, src_hbm, dst_hbm, zero_hbm, *rest):
        if with_cnt:
            (onehot_hbm, zero16_hbm, out_hbm, cnt_hbm,
             src_v, dst_v, rows_v, acc_sh, sem, sem_s,
             one_v, cacc_sh, sem_c) = rest
        else:
            (out_hbm, src_v, dst_v, rows_v, acc_sh, sem, sem_s) = rest

        cid = lax.axis_index("c")
        sid = lax.axis_index("s")
        wid = cid * _NS + sid

        pltpu.sync_copy(zero_hbm.at[pl.ds(sid * _STRIPE, _STRIPE)],
                        acc_sh.at[pl.ds(sid * _STRIPE, _STRIPE)])
        if with_cnt:
            pltpu.sync_copy(zero16_hbm.at[pl.ds(sid * _STRIPE, _STRIPE)],
                            cacc_sh.at[pl.ds(sid * _STRIPE, _STRIPE)])
            pltpu.sync_copy(onehot_hbm, one_v)

        @pl.when(sid == 0)
        def _():
            pltpu.sync_copy(zero_hbm.at[pl.ds(_NS * _STRIPE, _TAIL)],
                            acc_sh.at[pl.ds(_NS * _STRIPE, _TAIL)])
            if with_cnt:
                pltpu.sync_copy(zero16_hbm.at[pl.ds(_NS * _STRIPE, _TAIL)],
                                cacc_sh.at[pl.ds(_NS * _STRIPE, _TAIL)])

        plsc.subcore_barrier()

        def superblock(sb, carry):
            pltpu.sync_copy(src_hbm.at[wid, pl.ds(sb * _SB, _SB)], src_v)
            pltpu.sync_copy(dst_hbm.at[wid, pl.ds(sb * _SB, _SB)], dst_v)

            for b in range(_NB - 1):
                pltpu.async_copy(h_hbm.at[src_v.at[b]], rows_v.at[b], sem)

            def body(i, c):
                buf = lax.rem(i, _NB)
                pltpu.make_async_copy(h_hbm.at[src_v.at[i]], rows_v.at[buf],
                                      sem).wait()
                pltpu.async_copy(rows_v.at[buf], acc_sh.at[dst_v.at[i]],
                                 sem_s, add=True)
                if with_cnt:
                    pltpu.async_copy(one_v, cacc_sh.at[dst_v.at[i]],
                                     sem_c, add=True)

                @pl.when(i > 0)
                def _():
                    pltpu.make_async_copy(rows_v.at[buf],
                                          acc_sh.at[dst_v.at[i]],
                                          sem_s).wait()
                    if with_cnt:
                        pltpu.make_async_copy(one_v, cacc_sh.at[dst_v.at[i]],
                                              sem_c).wait()

                @pl.when(i + _NB - 1 < _SB)
                def _():
                    nxt = lax.rem(i + _NB - 1, _NB)
                    pltpu.async_copy(h_hbm.at[src_v.at[i + _NB - 1]],
                                     rows_v.at[nxt], sem)

                return c

            lax.fori_loop(0, _SB, body, 0)
            pltpu.make_async_copy(rows_v.at[0], acc_sh.at[dst_v.at[0]],
                                  sem_s).wait()
            if with_cnt:
                pltpu.make_async_copy(one_v, cacc_sh.at[dst_v.at[0]],
                                      sem_c).wait()
            return carry

        lax.fori_loop(0, _NCH // _SB, superblock, 0)

        plsc.subcore_barrier()

        pltpu.sync_copy(acc_sh.at[pl.ds(sid * _STRIPE, _STRIPE)],
                        out_hbm.at[cid, pl.ds(sid * _STRIPE, _STRIPE)])
        if with_cnt:
            pltpu.sync_copy(cacc_sh.at[pl.ds(sid * _STRIPE, _STRIPE)],
                            cnt_hbm.at[cid, pl.ds(sid * _STRIPE, _STRIPE)])

        @pl.when(sid == 0)
        def _():
            pltpu.sync_copy(acc_sh.at[pl.ds(_NS * _STRIPE, _TAIL)],
                            out_hbm.at[cid, pl.ds(_NS * _STRIPE, _TAIL)])
            if with_cnt:
                pltpu.sync_copy(cacc_sh.at[pl.ds(_NS * _STRIPE, _TAIL)],
                                cnt_hbm.at[cid, pl.ds(_NS * _STRIPE, _TAIL)])

    return sc_agg


def _dense1_body(acc_ref, cacc_ref, x_ref, wl_ref, bl_ref, wr_ref, g_ref,
                 be_ref, h_ref, cnt_ref):
    s = acc_ref[0] + acc_ref[1]
    cnt = cacc_ref[0, :, 0:1] + cacc_ref[1, :, 0:1]
    mean = s / jnp.maximum(cnt, 1.0)
    z = (lax.dot_general(mean, wl_ref[...], (((1,), (1,)), ((), ())),
                         preferred_element_type=jnp.float32)
         + lax.dot_general(x_ref[...], wr_ref[...], (((1,), (1,)), ((), ())),
                           preferred_element_type=jnp.float32)
         + bl_ref[...])
    mu = jnp.mean(z, axis=0)
    var = jnp.mean((z - mu) ** 2, axis=0)
    zn = (z - mu) / jnp.sqrt(var + 1e-5) * g_ref[...] + be_ref[...]
    h_ref[...] = jnp.maximum(zn, 0.0)
    cnt_ref[...] = cnt


def _dense2_body(acc_ref, cnt_ref, h_ref, wl_ref, bl_ref, wr_ref, g_ref,
                 be_ref, wc_ref, bc_ref, out_ref):
    s = acc_ref[0] + acc_ref[1]
    mean = s / jnp.maximum(cnt_ref[...], 1.0)
    z = (lax.dot_general(mean, wl_ref[...], (((1,), (1,)), ((), ())),
                         preferred_element_type=jnp.float32)
         + lax.dot_general(h_ref[...], wr_ref[...], (((1,), (1,)), ((), ())),
                           preferred_element_type=jnp.float32)
         + bl_ref[...])
    mu = jnp.mean(z, axis=0)
    var = jnp.mean((z - mu) ** 2, axis=0)
    zn = (z - mu) / jnp.sqrt(var + 1e-5) * g_ref[...] + be_ref[...]
    h2 = jnp.maximum(zn, 0.0)
    logits = (lax.dot_general(h2, wc_ref[...], (((1,), (1,)), ((), ())),
                              preferred_element_type=jnp.float32)
              + bc_ref[...])
    out_ref[...] = logits


def kernel(x, edge_index, W1l, b1l, W1r, g1, be1, W2l, b2l, W2r, g2, be2,
           Wc, bc):
    src = edge_index[0].reshape(_NW, _NCH, _CHUNK)
    dst = edge_index[1].reshape(_NW, _NCH, _CHUNK)

    zero_d = jnp.zeros((_N, _D), jnp.float32)
    zero16 = jnp.zeros((_N, _CW), jnp.float32)
    onehot = jnp.zeros((_CHUNK, _CW), jnp.float32).at[:, 0].set(1.0)

    acc1, cacc = _make_sc_agg(True)(x, src, dst, zero_d, onehot, zero16)

    h1, cnt = pl.pallas_call(
        _dense1_body,
        out_shape=(jax.ShapeDtypeStruct((_N, _D), jnp.float32),
                   jax.ShapeDtypeStruct((_N, 1), jnp.float32)),
    )(acc1, cacc, x, W1l, b1l, W1r, g1, be1)

    acc2 = _make_sc_agg(False)(h1, src, dst, zero_d)

    return pl.pallas_call(
        _dense2_body,
        out_shape=jax.ShapeDtypeStruct((_N, Wc.shape[0]), jnp.float32),
    )(acc2, cnt, h1, W2l, b2l, W2r, g2, be2, Wc, bc)


# speedup vs baseline: 13.3178x; 1.0330x over previous
import functools

import jax
import jax.numpy as jnp
from jax import lax
from jax.experimental import pallas as pl
from jax.experimental.pallas import tpu as pltpu
from jax.experimental.pallas import tpu_sc as plsc

_N = 10000
_E = 320000
_D = 128
_CW = 16

_NC = 2
_NS = 16
_NW = _NC * _NS
_EPW = _E // _NW
_CHUNK = 80
_NCH = _EPW // _CHUNK
_SB = 25
_NB = 3
_STRIPE = 624
_TAIL = _N - _NS * _STRIPE


@functools.lru_cache(maxsize=None)
def _make_sc_agg(with_cnt):
    mesh = plsc.VectorSubcoreMesh(core_axis_name="c", subcore_axis_name="s")

    out_type = [jax.ShapeDtypeStruct((_NC, _N, _D), jnp.float32)]
    scratch = [
        pltpu.VMEM((2, _SB, _CHUNK), jnp.int32),
        pltpu.VMEM((2, _SB, _CHUNK), jnp.int32),
        pltpu.VMEM((_NB, _CHUNK, _D), jnp.float32),
        pltpu.VMEM_SHARED((_N, _D), jnp.float32),
        pltpu.SemaphoreType.DMA,
        pltpu.SemaphoreType.DMA,
        pltpu.SemaphoreType.DMA,
    ]
    if with_cnt:
        out_type.append(jax.ShapeDtypeStruct((_NC, _N, _CW), jnp.float32))
        scratch += [
            pltpu.VMEM((_CHUNK, _CW), jnp.float32),
            pltpu.VMEM_SHARED((_N, _CW), jnp.float32),
            pltpu.SemaphoreType.DMA,
        ]

    @functools.partial(
        pl.kernel,
        out_type=tuple(out_type) if with_cnt else out_type[0],
        mesh=mesh,
        scratch_types=scratch,
        compiler_params=pltpu.CompilerParams(use_tc_tiling_on_sc=False),
    )
    def sc_agg(h_hbm, src_hbm, dst_hbm, zero_hbm, *rest):
        if with_cnt:
            (onehot_hbm, zero16_hbm, out_hbm, cnt_hbm,
             src_v, dst_v, rows_v, acc_sh, sem, sem_s, sem_i,
             one_v, cacc_sh, sem_c) = rest
        else:
            (out_hbm, src_v, dst_v, rows_v, acc_sh, sem, sem_s,
             sem_i) = rest

        cid = lax.axis_index("c")
        sid = lax.axis_index("s")
        wid = cid * _NS + sid

        pltpu.sync_copy(zero_hbm.at[pl.ds(sid * _STRIPE, _STRIPE)],
                        acc_sh.at[pl.ds(sid * _STRIPE, _STRIPE)])
        if with_cnt:
            pltpu.sync_copy(zero16_hbm.at[pl.ds(sid * _STRIPE, _STRIPE)],
                            cacc_sh.at[pl.ds(sid * _STRIPE, _STRIPE)])
            pltpu.sync_copy(onehot_hbm, one_v)

        @pl.when(sid == 0)
        def _():
            pltpu.sync_copy(zero_hbm.at[pl.ds(_NS * _STRIPE, _TAIL)],
                            acc_sh.at[pl.ds(_NS * _STRIPE, _TAIL)])
            if with_cnt:
                pltpu.sync_copy(zero16_hbm.at[pl.ds(_NS * _STRIPE, _TAIL)],
                                cacc_sh.at[pl.ds(_NS * _STRIPE, _TAIL)])

        pltpu.async_copy(src_hbm.at[wid, pl.ds(0, _SB)], src_v.at[0], sem_i)
        pltpu.async_copy(dst_hbm.at[wid, pl.ds(0, _SB)], dst_v.at[0], sem_i)

        plsc.subcore_barrier()

        _NSB = _NCH // _SB

        def superblock(sb, carry):
            b = lax.rem(sb, 2)
            pltpu.make_async_copy(src_hbm.at[wid, pl.ds(sb * _SB, _SB)],
                                  src_v.at[b], sem_i).wait()
            pltpu.make_async_copy(dst_hbm.at[wid, pl.ds(sb * _SB, _SB)],
                                  dst_v.at[b], sem_i).wait()

            @pl.when(sb + 1 < _NSB)
            def _():
                nb = lax.rem(sb + 1, 2)
                pltpu.async_copy(src_hbm.at[wid, pl.ds((sb + 1) * _SB, _SB)],
                                 src_v.at[nb], sem_i)
                pltpu.async_copy(dst_hbm.at[wid, pl.ds((sb + 1) * _SB, _SB)],
                                 dst_v.at[nb], sem_i)

            for p in range(_NB - 1):
                pltpu.async_copy(h_hbm.at[src_v.at[b, p]], rows_v.at[p], sem)

            def body(i, c):
                buf = lax.rem(i, _NB)
                pltpu.make_async_copy(h_hbm.at[src_v.at[b, i]],
                                      rows_v.at[buf], sem).wait()
                pltpu.async_copy(rows_v.at[buf], acc_sh.at[dst_v.at[b, i]],
                                 sem_s, add=True)
                if with_cnt:
                    pltpu.async_copy(one_v, cacc_sh.at[dst_v.at[b, i]],
                                     sem_c, add=True)

                @pl.when(i > 0)
                def _():
                    pltpu.make_async_copy(rows_v.at[buf],
                                          acc_sh.at[dst_v.at[b, i]],
                                          sem_s).wait()
                    if with_cnt:
                        pltpu.make_async_copy(one_v,
                                              cacc_sh.at[dst_v.at[b, i]],
                                              sem_c).wait()

                @pl.when(i + _NB - 1 < _SB)
                def _():
                    nxt = lax.rem(i + _NB - 1, _NB)
                    pltpu.async_copy(h_hbm.at[src_v.at[b, i + _NB - 1]],
                                     rows_v.at[nxt], sem)

                return c

            lax.fori_loop(0, _SB, body, 0)
            pltpu.make_async_copy(rows_v.at[0], acc_sh.at[dst_v.at[0, 0]],
                                  sem_s).wait()
            if with_cnt:
                pltpu.make_async_copy(one_v, cacc_sh.at[dst_v.at[0, 0]],
                                      sem_c).wait()
            return carry

        lax.fori_loop(0, _NSB, superblock, 0)

        plsc.subcore_barrier()

        pltpu.sync_copy(acc_sh.at[pl.ds(sid * _STRIPE, _STRIPE)],
                        out_hbm.at[cid, pl.ds(sid * _STRIPE, _STRIPE)])
        if with_cnt:
            pltpu.sync_copy(cacc_sh.at[pl.ds(sid * _STRIPE, _STRIPE)],
                            cnt_hbm.at[cid, pl.ds(sid * _STRIPE, _STRIPE)])

        @pl.when(sid == 0)
        def _():
            pltpu.sync_copy(acc_sh.at[pl.ds(_NS * _STRIPE, _TAIL)],
                            out_hbm.at[cid, pl.ds(_NS * _STRIPE, _TAIL)])
            if with_cnt:
                pltpu.sync_copy(cacc_sh.at[pl.ds(_NS * _STRIPE, _TAIL)],
                                cnt_hbm.at[cid, pl.ds(_NS * _STRIPE, _TAIL)])

    return sc_agg


def _dense1_body(acc_ref, cacc_ref, x_ref, wl_ref, bl_ref, wr_ref, g_ref,
                 be_ref, h_ref, cnt_ref):
    s = acc_ref[0] + acc_ref[1]
    cnt = cacc_ref[0, :, 0:1] + cacc_ref[1, :, 0:1]
    mean = s / jnp.maximum(cnt, 1.0)
    z = (lax.dot_general(mean, wl_ref[...], (((1,), (1,)), ((), ())),
                         preferred_element_type=jnp.float32)
         + lax.dot_general(x_ref[...], wr_ref[...], (((1,), (1,)), ((), ())),
                           preferred_element_type=jnp.float32)
         + bl_ref[...])
    mu = jnp.mean(z, axis=0)
    var = jnp.mean((z - mu) ** 2, axis=0)
    zn = (z - mu) / jnp.sqrt(var + 1e-5) * g_ref[...] + be_ref[...]
    h_ref[...] = jnp.maximum(zn, 0.0)
    cnt_ref[...] = cnt


def _dense2_body(acc_ref, cnt_ref, h_ref, wl_ref, bl_ref, wr_ref, g_ref,
                 be_ref, wc_ref, bc_ref, out_ref):
    s = acc_ref[0] + acc_ref[1]
    mean = s / jnp.maximum(cnt_ref[...], 1.0)
    z = (lax.dot_general(mean, wl_ref[...], (((1,), (1,)), ((), ())),
                         preferred_element_type=jnp.float32)
         + lax.dot_general(h_ref[...], wr_ref[...], (((1,), (1,)), ((), ())),
                           preferred_element_type=jnp.float32)
         + bl_ref[...])
    mu = jnp.mean(z, axis=0)
    var = jnp.mean((z - mu) ** 2, axis=0)
    zn = (z - mu) / jnp.sqrt(var + 1e-5) * g_ref[...] + be_ref[...]
    h2 = jnp.maximum(zn, 0.0)
    logits = (lax.dot_general(h2, wc_ref[...], (((1,), (1,)), ((), ())),
                              preferred_element_type=jnp.float32)
              + bc_ref[...])
    out_ref[...] = logits


def kernel(x, edge_index, W1l, b1l, W1r, g1, be1, W2l, b2l, W2r, g2, be2,
           Wc, bc):
    src = edge_index[0].reshape(_NW, _NCH, _CHUNK)
    dst = edge_index[1].reshape(_NW, _NCH, _CHUNK)

    zero_d = jnp.zeros((_N, _D), jnp.float32)
    zero16 = jnp.zeros((_N, _CW), jnp.float32)
    onehot = jnp.zeros((_CHUNK, _CW), jnp.float32).at[:, 0].set(1.0)

    acc1, cacc = _make_sc_agg(True)(x, src, dst, zero_d, onehot, zero16)

    h1, cnt = pl.pallas_call(
        _dense1_body,
        out_shape=(jax.ShapeDtypeStruct((_N, _D), jnp.float32),
                   jax.ShapeDtypeStruct((_N, 1), jnp.float32)),
    )(acc1, cacc, x, W1l, b1l, W1r, g1, be1)

    acc2 = _make_sc_agg(False)(h1, src, dst, zero_d)

    return pl.pallas_call(
        _dense2_body,
        out_shape=jax.ShapeDtypeStruct((_N, Wc.shape[0]), jnp.float32),
    )(acc2, cnt, h1, W2l, b2l, W2r, g2, be2, Wc, bc)


# speedup vs baseline: 13.6597x; 1.0257x over previous
import functools

import jax
import jax.numpy as jnp
from jax import lax
from jax.experimental import pallas as pl
from jax.experimental.pallas import tpu as pltpu
from jax.experimental.pallas import tpu_sc as plsc

_N = 10000
_E = 320000
_D = 128
_CW = 16

_NC = 2
_NS = 16
_NW = _NC * _NS
_EPW = _E // _NW
_CHUNK = 80
_NCH = _EPW // _CHUNK
_SB = 25
_NB = 3
_STRIPE = 624
_TAIL = _N - _NS * _STRIPE


@functools.lru_cache(maxsize=None)
def _make_sc_agg(with_cnt):
    mesh = plsc.VectorSubcoreMesh(core_axis_name="c", subcore_axis_name="s")

    out_type = [jax.ShapeDtypeStruct((_NC, _N, _D), jnp.float32)]
    scratch = [
        pltpu.VMEM((2, _SB, _CHUNK), jnp.int32),
        pltpu.VMEM((2, _SB, _CHUNK), jnp.int32),
        pltpu.VMEM((_NB, _CHUNK, _D), jnp.float32),
        pltpu.VMEM_SHARED((_N, _D), jnp.float32),
        pltpu.SemaphoreType.DMA,
        pltpu.SemaphoreType.DMA,
        pltpu.SemaphoreType.DMA,
    ]
    if with_cnt:
        out_type.append(jax.ShapeDtypeStruct((_NC, _N, _CW), jnp.float32))
        scratch += [
            pltpu.VMEM((_CHUNK, _CW), jnp.float32),
            pltpu.VMEM_SHARED((_N, _CW), jnp.float32),
            pltpu.SemaphoreType.DMA,
        ]

    @functools.partial(
        pl.kernel,
        out_type=tuple(out_type) if with_cnt else out_type[0],
        mesh=mesh,
        scratch_types=scratch,
        compiler_params=pltpu.CompilerParams(use_tc_tiling_on_sc=False),
    )
    def sc_agg(h_hbm, src_hbm, dst_hbm, zero_hbm, *rest):
        if with_cnt:
            (onehot_hbm, zero16_hbm, out_hbm, cnt_hbm,
             src_v, dst_v, rows_v, acc_sh, sem, sem_s, sem_i,
             one_v, cacc_sh, sem_c) = rest
        else:
            (out_hbm, src_v, dst_v, rows_v, acc_sh, sem, sem_s,
             sem_i) = rest

        cid = lax.axis_index("c")
        sid = lax.axis_index("s")
        wid = cid * _NS + sid

        pltpu.sync_copy(zero_hbm.at[pl.ds(sid * _STRIPE, _STRIPE)],
                        acc_sh.at[pl.ds(sid * _STRIPE, _STRIPE)])
        if with_cnt:
            pltpu.sync_copy(zero16_hbm.at[pl.ds(sid * _STRIPE, _STRIPE)],
                            cacc_sh.at[pl.ds(sid * _STRIPE, _STRIPE)])
            pltpu.sync_copy(onehot_hbm, one_v)

        @pl.when(sid == 0)
        def _():
            pltpu.sync_copy(zero_hbm.at[pl.ds(_NS * _STRIPE, _TAIL)],
                            acc_sh.at[pl.ds(_NS * _STRIPE, _TAIL)])
            if with_cnt:
                pltpu.sync_copy(zero16_hbm.at[pl.ds(_NS * _STRIPE, _TAIL)],
                                cacc_sh.at[pl.ds(_NS * _STRIPE, _TAIL)])

        pltpu.sync_copy(src_hbm.at[wid, pl.ds(0, _SB)], src_v.at[0])
        pltpu.sync_copy(dst_hbm.at[wid, pl.ds(0, _SB)], dst_v.at[0])
        for p in range(_NB - 1):
            pltpu.async_copy(h_hbm.at[src_v.at[0, p]], rows_v.at[p], sem)

        plsc.subcore_barrier()

        def body(i, c):
            r = lax.rem(i, _SB)
            sbuf = lax.rem(lax.div(i, _SB), 2)
            buf = lax.rem(i, _NB)

            pltpu.make_async_copy(h_hbm.at[src_v.at[sbuf, r]],
                                  rows_v.at[buf], sem).wait()
            pltpu.async_copy(rows_v.at[buf], acc_sh.at[dst_v.at[sbuf, r]],
                             sem_s, add=True)
            if with_cnt:
                pltpu.async_copy(one_v, cacc_sh.at[dst_v.at[sbuf, r]],
                                 sem_c, add=True)

            @pl.when(i > 0)
            def _():
                pltpu.make_async_copy(rows_v.at[buf],
                                      acc_sh.at[dst_v.at[sbuf, r]],
                                      sem_s).wait()
                if with_cnt:
                    pltpu.make_async_copy(one_v, cacc_sh.at[dst_v.at[sbuf, r]],
                                          sem_c).wait()

            @pl.when((r == 0) & (i + _SB < _NCH))
            def _():
                nb = 1 - sbuf
                pltpu.async_copy(src_hbm.at[wid, pl.ds(i + _SB, _SB)],
                                 src_v.at[nb], sem_i)
                pltpu.async_copy(dst_hbm.at[wid, pl.ds(i + _SB, _SB)],
                                 dst_v.at[nb], sem_i)

            @pl.when((r == _SB - 2) & (i + 2 < _NCH))
            def _():
                pltpu.make_async_copy(src_hbm.at[wid, pl.ds(0, _SB)],
                                      src_v.at[0], sem_i).wait()
                pltpu.make_async_copy(dst_hbm.at[wid, pl.ds(0, _SB)],
                                      dst_v.at[0], sem_i).wait()

            @pl.when(i + 2 < _NCH)
            def _():
                j = i + 2
                rj = lax.rem(j, _SB)
                sj = lax.rem(lax.div(j, _SB), 2)
                pltpu.async_copy(h_hbm.at[src_v.at[sj, rj]],
                                 rows_v.at[lax.rem(j, _NB)], sem)

            return c

        lax.fori_loop(0, _NCH, body, 0)

        pltpu.make_async_copy(rows_v.at[0], acc_sh.at[dst_v.at[0, 0]],
                              sem_s).wait()
        if with_cnt:
            pltpu.make_async_copy(one_v, cacc_sh.at[dst_v.at[0, 0]],
                                  sem_c).wait()

        plsc.subcore_barrier()

        pltpu.sync_copy(acc_sh.at[pl.ds(sid * _STRIPE, _STRIPE)],
                        out_hbm.at[cid, pl.ds(sid * _STRIPE, _STRIPE)])
        if with_cnt:
            pltpu.sync_copy(cacc_sh.at[pl.ds(sid * _STRIPE, _STRIPE)],
                            cnt_hbm.at[cid, pl.ds(sid * _STRIPE, _STRIPE)])

        @pl.when(sid == 0)
        def _():
            pltpu.sync_copy(acc_sh.at[pl.ds(_NS * _STRIPE, _TAIL)],
                            out_hbm.at[cid, pl.ds(_NS * _STRIPE, _TAIL)])
            if with_cnt:
                pltpu.sync_copy(cacc_sh.at[pl.ds(_NS * _STRIPE, _TAIL)],
                                cnt_hbm.at[cid, pl.ds(_NS * _STRIPE, _TAIL)])

    return sc_agg


def _dense1_body(acc_ref, cacc_ref, x_ref, wl_ref, bl_ref, wr_ref, g_ref,
                 be_ref, h_ref, cnt_ref):
    s = acc_ref[0] + acc_ref[1]
    cnt = cacc_ref[0, :, 0:1] + cacc_ref[1, :, 0:1]
    mean = s / jnp.maximum(cnt, 1.0)
    z = (lax.dot_general(mean, wl_ref[...], (((1,), (1,)), ((), ())),
                         preferred_element_type=jnp.float32)
         + lax.dot_general(x_ref[...], wr_ref[...], (((1,), (1,)), ((), ())),
                           preferred_element_type=jnp.float32)
         + bl_ref[...])
    mu = jnp.mean(z, axis=0)
    var = jnp.mean((z - mu) ** 2, axis=0)
    zn = (z - mu) / jnp.sqrt(var + 1e-5) * g_ref[...] + be_ref[...]
    h_ref[...] = jnp.maximum(zn, 0.0)
    cnt_ref[...] = cnt


def _dense2_body(acc_ref, cnt_ref, h_ref, wl_ref, bl_ref, wr_ref, g_ref,
                 be_ref, wc_ref, bc_ref, out_ref):
    s = acc_ref[0] + acc_ref[1]
    mean = s / jnp.maximum(cnt_ref[...], 1.0)
    z = (lax.dot_general(mean, wl_ref[...], (((1,), (1,)), ((), ())),
                         preferred_element_type=jnp.float32)
         + lax.dot_general(h_ref[...], wr_ref[...], (((1,), (1,)), ((), ())),
                           preferred_element_type=jnp.float32)
         + bl_ref[...])
    mu = jnp.mean(z, axis=0)
    var = jnp.mean((z - mu) ** 2, axis=0)
    zn = (z - mu) / jnp.sqrt(var + 1e-5) * g_ref[...] + be_ref[...]
    h2 = jnp.maximum(zn, 0.0)
    logits = (lax.dot_general(h2, wc_ref[...], (((1,), (1,)), ((), ())),
                              preferred_element_type=jnp.float32)
              + bc_ref[...])
    out_ref[...] = logits


def kernel(x, edge_index, W1l, b1l, W1r, g1, be1, W2l, b2l, W2r, g2, be2,
           Wc, bc):
    src = edge_index[0].reshape(_NW, _NCH, _CHUNK)
    dst = edge_index[1].reshape(_NW, _NCH, _CHUNK)

    zero_d = jnp.zeros((_N, _D), jnp.float32)
    zero16 = jnp.zeros((_N, _CW), jnp.float32)
    onehot = jnp.zeros((_CHUNK, _CW), jnp.float32).at[:, 0].set(1.0)

    acc1, cacc = _make_sc_agg(True)(x, src, dst, zero_d, onehot, zero16)

    h1, cnt = pl.pallas_call(
        _dense1_body,
        out_shape=(jax.ShapeDtypeStruct((_N, _D), jnp.float32),
                   jax.ShapeDtypeStruct((_N, 1), jnp.float32)),
    )(acc1, cacc, x, W1l, b1l, W1r, g1, be1)

    acc2 = _make_sc_agg(False)(h1, src, dst, zero_d)

    return pl.pallas_call(
        _dense2_body,
        out_shape=jax.ShapeDtypeStruct((_N, Wc.shape[0]), jnp.float32),
    )(acc2, cnt, h1, W2l, b2l, W2r, g2, be2, Wc, bc)
